# Initial kernel scaffold; baseline (speedup 1.0000x reference)
import jax
import jax.numpy as jnp
from jax.experimental import pallas as pl


def kernel(x, edge_index, edge_weight, batch, W_rel1, b_rel1, W_root1, W_rel2, b_rel2, W_root2, W_rel3, b_rel3, W_root3, p1, p2, p3, W_lin1, b_lin1, W_lin3, b_lin3):
    raise NotImplementedError("write your pallas kernel here")


# speedup vs baseline: 1.0349x; 1.0349x over previous
import jax
import jax.numpy as jnp
from jax import lax
from jax.experimental import pallas as pl
from jax.experimental.pallas import tpu as pltpu

N_NODES = 10000
N_EDGES = 320000
D_FEAT = 128
HID = 32
NUM_GRAPHS = 16
NUM_CLASSES = 8
RATIO = 0.3


def _mm2_body(h_ref, wa_ref, wb_ref, oa_ref, ob_ref):
    h = h_ref[...]
    oa_ref[...] = jnp.dot(h, wa_ref[...], preferred_element_type=jnp.float32)
    ob_ref[...] = jnp.dot(h, wb_ref[...], preferred_element_type=jnp.float32)


def _mm2(h, wa, wb):
    n = h.shape[0]
    k = wa.shape[1]
    return pl.pallas_call(
        _mm2_body,
        out_shape=(jax.ShapeDtypeStruct((n, k), jnp.float32),
                   jax.ShapeDtypeStruct((n, k), jnp.float32)),
    )(h, wa, wb)


def _post_body(agg_ref, hroot_ref, b_ref, p_ref, h_ref, s_ref):
    h = jnp.maximum((agg_ref[...] + b_ref[...]) + hroot_ref[...], 0.0)
    p = p_ref[...]
    nrm = jnp.sqrt(jnp.sum(p * p))
    h_ref[...] = h
    s_ref[...] = jnp.tanh(
        jnp.sum(h * p, axis=1, keepdims=True) / nrm)


def _post(agg, hroot, b, p):
    n = agg.shape[0]
    return pl.pallas_call(
        _post_body,
        out_shape=(jax.ShapeDtypeStruct((n, HID), jnp.float32),
                   jax.ShapeDtypeStruct((n, 1), jnp.float32)),
    )(agg, hroot, b.reshape(1, HID), p.reshape(1, HID))


def _readout_body(cnt_ref, st_ref, h_ref, out_ref):
    g = pl.program_id(0)
    h = h_ref[...]
    n = h.shape[0]
    rows = lax.broadcasted_iota(jnp.int32, (n, 1), 0)
    st = st_ref[g]
    cnt = cnt_ref[g]
    m = (rows >= st) & (rows < st + cnt)
    gmax = jnp.max(jnp.where(m, h, -jnp.inf), axis=0, keepdims=True)
    gsum = jnp.sum(jnp.where(m, h, 0.0), axis=0, keepdims=True)
    out_ref[pl.ds(g, 1), 0:HID] = gmax
    out_ref[pl.ds(g, 1), HID:2 * HID] = gsum / cnt.astype(jnp.float32)


def _readout(h, counts, starts):
    n = h.shape[0]
    return pl.pallas_call(
        _readout_body,
        grid=(NUM_GRAPHS,),
        in_specs=[pl.BlockSpec(memory_space=pltpu.SMEM),
                  pl.BlockSpec(memory_space=pltpu.SMEM),
                  pl.BlockSpec((n, HID), lambda g: (0, 0))],
        out_specs=pl.BlockSpec((NUM_GRAPHS, 2 * HID), lambda g: (0, 0)),
        out_shape=jax.ShapeDtypeStruct((NUM_GRAPHS, 2 * HID), jnp.float32),
    )(counts, starts, h)


def _final_body(r1_ref, r2_ref, r3_ref, w1_ref, b1_ref, w3_ref, b3_ref, out_ref):
    z = r1_ref[...] + r2_ref[...] + r3_ref[...]
    z = jnp.maximum(
        jnp.dot(z, w1_ref[...], preferred_element_type=jnp.float32) + b1_ref[...], 0.0)
    z = jnp.dot(z, w3_ref[...], preferred_element_type=jnp.float32) + b3_ref[...]
    m = jnp.max(z, axis=-1, keepdims=True)
    lse = jnp.log(jnp.sum(jnp.exp(z - m), axis=-1, keepdims=True)) + m
    out_ref[...] = z - lse


def _final(r1, r2, r3, w1, b1, w3, b3):
    return pl.pallas_call(
        _final_body,
        out_shape=jax.ShapeDtypeStruct((NUM_GRAPHS, NUM_CLASSES), jnp.float32),
    )(r1, r2, r3, w1, b1.reshape(1, HID), w3, b3.reshape(1, NUM_CLASSES))


def _topk_sel(score, counts, starts, batch_ids, valid):
    n = score.shape[0]
    idx = jnp.arange(n, dtype=jnp.int32)
    pos = idx - starts[batch_ids]
    row = jnp.where(valid, batch_ids, NUM_GRAPHS)
    col = jnp.where(valid, pos, n)
    dense = jnp.full((NUM_GRAPHS, n), -jnp.inf, dtype=score.dtype)
    dense = dense.at[row, col].set(score, mode="drop")
    order = jnp.argsort(-dense, axis=1).astype(jnp.int32)
    rden = 10
    rnum = int(round(float(RATIO) * rden))
    k = jnp.minimum(jnp.maximum((rnum * counts + rden - 1) // rden, 1),
                    jnp.maximum(counts, 1))
    new_starts = jnp.concatenate(
        [jnp.zeros((1,), jnp.int32), jnp.cumsum(k)[:-1].astype(jnp.int32)])
    j = jnp.arange(n, dtype=jnp.int32)[None, :]
    sel = j < k[:, None]
    tgt = jnp.where(sel, new_starts[:, None] + j, n)
    vals = starts[:, None] + order
    perm = jnp.zeros((n,), dtype=jnp.int32).at[tgt.ravel()].set(
        vals.ravel(), mode="drop")
    gid = jnp.broadcast_to(
        jnp.arange(NUM_GRAPHS, dtype=jnp.int32)[:, None], (NUM_GRAPHS, n))
    new_batch = jnp.full((n,), NUM_GRAPHS - 1, dtype=jnp.int32).at[
        tgt.ravel()].set(gid.ravel(), mode="drop")
    new_valid = idx < k.sum()
    return perm, k, new_batch, new_valid, new_starts


def _filter_e(src, dst, emask, perm, new_valid, n):
    safe = jnp.where(new_valid, perm, n)
    inv = jnp.full((n,), -1, dtype=jnp.int32).at[safe].set(
        jnp.arange(n, dtype=jnp.int32), mode="drop")
    s = inv[src]
    d = inv[dst]
    ok = (s >= 0) & (d >= 0)
    new_mask = emask * ok.astype(emask.dtype)
    return jnp.where(ok, s, 0), jnp.where(ok, d, 0), new_mask


def kernel(x, edge_index, edge_weight, batch, W_rel1, b_rel1, W_root1,
           W_rel2, b_rel2, W_root2, W_rel3, b_rel3, W_root3, p1, p2, p3,
           W_lin1, b_lin1, W_lin3, b_lin3):
    n = N_NODES
    batch_ids = batch.astype(jnp.int32)
    src = jnp.asarray(edge_index[0])
    dst = jnp.asarray(edge_index[1])
    emask = jnp.ones((N_EDGES,), dtype=jnp.float32)
    valid = jnp.ones((n,), dtype=bool)
    counts = jax.ops.segment_sum(
        jnp.ones((n,), jnp.int32), batch_ids, num_segments=NUM_GRAPHS)
    starts = jnp.concatenate(
        [jnp.zeros((1,), jnp.int32), jnp.cumsum(counts)[:-1].astype(jnp.int32)])

    h = x
    readouts = []
    layers = [(W_rel1, b_rel1, W_root1, p1),
              (W_rel2, b_rel2, W_root2, p2),
              (W_rel3, b_rel3, W_root3, p3)]
    for (Wr, br, Wroot, p) in layers:
        hr, hroot = _mm2(h, Wr, Wroot)
        agg = jnp.zeros((n, HID), jnp.float32).at[dst].add(
            hr[src] * emask[:, None])
        h2, s2 = _post(agg, hroot, br, p)
        score = s2[:, 0]
        perm, k, new_batch, new_valid, new_starts = _topk_sel(
            score, counts, starts, batch_ids, valid)
        h = h2[perm] * score[perm][:, None] * new_valid.astype(jnp.float32)[:, None]
        src, dst, emask = _filter_e(src, dst, emask, perm, new_valid, n)
        batch_ids = new_batch
        valid = new_valid
        counts = k
        starts = new_starts
        readouts.append(_readout(h, counts, starts))

    return _final(readouts[0], readouts[1], readouts[2],
                  W_lin1, b_lin1, W_lin3, b_lin3)


# speedup vs baseline: 1.2073x; 1.1665x over previous
import jax
import jax.numpy as jnp
from jax import lax
from jax.experimental import pallas as pl
from jax.experimental.pallas import tpu as pltpu

N_NODES = 10000
N_EDGES = 320000
D_FEAT = 128
HID = 32
NUM_GRAPHS = 16
NUM_CLASSES = 8
RATIO = 0.3


def _convpost_body(agg_ref, h_ref, wr_ref, b_ref, wro_ref, p_ref, pn_ref,
                   h2_ref, s_ref):
    z = (jnp.dot(agg_ref[...], wr_ref[...],
                 preferred_element_type=jnp.float32) + b_ref[...]) \
        + jnp.dot(h_ref[...], wro_ref[...], preferred_element_type=jnp.float32)
    h2 = jnp.maximum(z, 0.0)
    h2_ref[...] = h2
    q = jnp.dot(h2, p_ref[...], preferred_element_type=jnp.float32)
    s_ref[...] = jnp.tanh(q / pn_ref[0, 0])


def _convpost(agg, h, wr, b, wro, p, pnorm):
    n = agg.shape[0]
    return pl.pallas_call(
        _convpost_body,
        in_specs=[pl.BlockSpec(memory_space=pltpu.VMEM)] * 6
        + [pl.BlockSpec(memory_space=pltpu.SMEM)],
        out_shape=(jax.ShapeDtypeStruct((n, HID), jnp.float32),
                   jax.ShapeDtypeStruct((n, 1), jnp.float32)),
    )(agg, h, wr, b.reshape(1, HID), wro, p.reshape(HID, 1),
      pnorm.reshape(1, 1))


def _readout_body(cnt_ref, st_ref, h_ref, out_ref):
    g = pl.program_id(0)
    h = h_ref[...]
    n = h.shape[0]
    rows = lax.broadcasted_iota(jnp.int32, (n, 1), 0)
    st = st_ref[g]
    cnt = cnt_ref[g]
    m = (rows >= st) & (rows < st + cnt)
    gmax = jnp.max(jnp.where(m, h, -jnp.inf), axis=0, keepdims=True)
    gsum = jnp.sum(jnp.where(m, h, 0.0), axis=0, keepdims=True)
    out_ref[pl.ds(g, 1), 0:HID] = gmax
    out_ref[pl.ds(g, 1), HID:2 * HID] = gsum / cnt.astype(jnp.float32)


def _readout(h, counts, starts):
    n = h.shape[0]
    return pl.pallas_call(
        _readout_body,
        grid=(NUM_GRAPHS,),
        in_specs=[pl.BlockSpec(memory_space=pltpu.SMEM),
                  pl.BlockSpec(memory_space=pltpu.SMEM),
                  pl.BlockSpec((n, HID), lambda g: (0, 0))],
        out_specs=pl.BlockSpec((NUM_GRAPHS, 2 * HID), lambda g: (0, 0)),
        out_shape=jax.ShapeDtypeStruct((NUM_GRAPHS, 2 * HID), jnp.float32),
    )(counts, starts, h)


def _final_body(r1_ref, r2_ref, r3_ref, w1_ref, b1_ref, w3_ref, b3_ref, out_ref):
    z = r1_ref[...] + r2_ref[...] + r3_ref[...]
    z = jnp.maximum(
        jnp.dot(z, w1_ref[...], preferred_element_type=jnp.float32) + b1_ref[...], 0.0)
    z = jnp.dot(z, w3_ref[...], preferred_element_type=jnp.float32) + b3_ref[...]
    m = jnp.max(z, axis=-1, keepdims=True)
    lse = jnp.log(jnp.sum(jnp.exp(z - m), axis=-1, keepdims=True)) + m
    out_ref[...] = z - lse


def _final(r1, r2, r3, w1, b1, w3, b3):
    return pl.pallas_call(
        _final_body,
        out_shape=jax.ShapeDtypeStruct((NUM_GRAPHS, NUM_CLASSES), jnp.float32),
    )(r1, r2, r3, w1, b1.reshape(1, HID), w3, b3.reshape(1, NUM_CLASSES))


def _thresh_body(key_ref, st_ref, cnt_ref, k_ref, t_ref, cgt_ref):
    n = key_ref.shape[1]
    K = key_ref[...]
    st = st_ref[...]
    cnt = cnt_ref[...]
    kk = k_ref[...]
    cols = lax.broadcasted_iota(jnp.int32, (NUM_GRAPHS, n), 1)
    R = (cols >= st) & (cols < st + cnt)

    def body(_, carry):
        lo, hi = carry
        mid = (lo >> 1) + (hi >> 1) + (lo & hi & 1)
        f = jnp.sum(jnp.where(R & (K > mid), 1, 0), axis=1, keepdims=True)
        p = f < kk
        return (jnp.where(p, lo, mid), jnp.where(p, mid, hi))

    lo0 = jnp.full((NUM_GRAPHS, 1), jnp.iinfo(jnp.int32).min, jnp.int32)
    hi0 = jnp.full((NUM_GRAPHS, 1), jnp.iinfo(jnp.int32).max, jnp.int32)
    _, t = lax.fori_loop(0, 32, body, (lo0, hi0))
    t_ref[...] = t
    cgt_ref[...] = jnp.sum(jnp.where(R & (K > t), 1, 0), axis=1, keepdims=True)


def _thresh(key, starts, counts, k):
    n = key.shape[0]
    return pl.pallas_call(
        _thresh_body,
        out_shape=(jax.ShapeDtypeStruct((NUM_GRAPHS, 1), jnp.int32),
                   jax.ShapeDtypeStruct((NUM_GRAPHS, 1), jnp.int32)),
    )(key.reshape(1, n), starts.reshape(NUM_GRAPHS, 1),
      counts.reshape(NUM_GRAPHS, 1), k.reshape(NUM_GRAPHS, 1))


def _topk_sel(score, counts, starts, batch_ids, valid):
    n = score.shape[0]
    idx = jnp.arange(n, dtype=jnp.int32)
    b = lax.bitcast_convert_type(score, jnp.int32)
    key = b ^ ((b >> 31) & jnp.int32(0x7FFFFFFF))
    rden = 10
    rnum = int(round(float(RATIO) * rden))
    k = jnp.minimum(jnp.maximum((rnum * counts + rden - 1) // rden, 1),
                    jnp.maximum(counts, 1))
    t2, cgt2 = _thresh(key, starts, counts, k)
    t = t2[:, 0]
    need = k - cgt2[:, 0]
    tpn = t[batch_ids]
    npn = need[batch_ids]
    gt = valid & (key > tpn)
    eq = valid & (key == tpn)
    eqi = eq.astype(jnp.int32)
    ex = jnp.cumsum(eqi) - eqi
    eqrank = ex - ex[starts][batch_ids]
    sel = gt | (eq & (eqrank < npn))
    ranks = jnp.cumsum(sel.astype(jnp.int32))
    tgt = jnp.where(sel, ranks - 1, n)
    perm = jnp.zeros((n,), dtype=jnp.int32).at[tgt].set(idx, mode="drop")
    new_batch = jnp.full((n,), NUM_GRAPHS - 1, dtype=jnp.int32).at[tgt].set(
        batch_ids, mode="drop")
    new_starts = jnp.concatenate(
        [jnp.zeros((1,), jnp.int32), jnp.cumsum(k)[:-1].astype(jnp.int32)])
    gids = jnp.arange(NUM_GRAPHS, dtype=jnp.int32)
    perm = perm.at[new_starts].set(
        jnp.where(counts == 0, starts, perm[new_starts]))
    new_batch = new_batch.at[new_starts].set(
        jnp.where(counts == 0, gids, new_batch[new_starts]))
    new_valid = idx < k.sum()
    return perm, k, new_batch, new_valid, new_starts


def _filter_e(src, dst, emask, perm, new_valid, n):
    safe = jnp.where(new_valid, perm, n)
    inv = jnp.full((n,), -1, dtype=jnp.int32).at[safe].set(
        jnp.arange(n, dtype=jnp.int32), mode="drop")
    s = inv[src]
    d = inv[dst]
    ok = (s >= 0) & (d >= 0)
    new_mask = emask * ok.astype(emask.dtype)
    return jnp.where(ok, s, 0), jnp.where(ok, d, 0), new_mask


def kernel(x, edge_index, edge_weight, batch, W_rel1, b_rel1, W_root1,
           W_rel2, b_rel2, W_root2, W_rel3, b_rel3, W_root3, p1, p2, p3,
           W_lin1, b_lin1, W_lin3, b_lin3):
    n = N_NODES
    batch_ids = batch.astype(jnp.int32)
    src = jnp.asarray(edge_index[0])
    dst = jnp.asarray(edge_index[1])
    emask = jnp.ones((N_EDGES,), dtype=jnp.float32)
    valid = jnp.ones((n,), dtype=bool)
    counts = jax.ops.segment_sum(
        jnp.ones((n,), jnp.int32), batch_ids, num_segments=NUM_GRAPHS)
    starts = jnp.concatenate(
        [jnp.zeros((1,), jnp.int32), jnp.cumsum(counts)[:-1].astype(jnp.int32)])

    h = x
    readouts = []
    layers = [(W_rel1, b_rel1, W_root1, p1),
              (W_rel2, b_rel2, W_root2, p2),
              (W_rel3, b_rel3, W_root3, p3)]
    for (Wr, br, Wroot, p) in layers:
        agg = jnp.zeros(h.shape, jnp.float32).at[dst].add(
            h[src] * emask[:, None])
        h2, s2 = _convpost(agg, h, Wr, br, Wroot, p, jnp.linalg.norm(p))
        score = s2[:, 0]
        perm, k, new_batch, new_valid, new_starts = _topk_sel(
            score, counts, starts, batch_ids, valid)
        h = h2[perm] * score[perm][:, None] * new_valid.astype(jnp.float32)[:, None]
        src, dst, emask = _filter_e(src, dst, emask, perm, new_valid, n)
        batch_ids = new_batch
        valid = new_valid
        counts = k
        starts = new_starts
        readouts.append(_readout(h, counts, starts))

    return _final(readouts[0], readouts[1], readouts[2],
                  W_lin1, b_lin1, W_lin3, b_lin3)
